# baseline (speedup 1.0000x reference)
import functools

import jax
import jax.numpy as jnp
from jax import lax
from jax.experimental import pallas as pl
from jax.experimental.pallas import tpu as pltpu
from jax.experimental.pallas import tpu_sc as plsc

NC = 2
NS = 16
NW = NC * NS
GROUP = 128
K = 5


@functools.lru_cache(maxsize=None)
def _build(total_rows: int, vocab: int, embed: int):
    assert total_rows % (NW * GROUP) == 0
    n_groups = total_rows // (NW * GROUP)
    assert n_groups % K == 0
    n_halves = n_groups // K
    assert n_halves >= 2
    mesh = plsc.VectorSubcoreMesh(core_axis_name="c", subcore_axis_name="s")

    @functools.partial(
        pl.kernel,
        mesh=mesh,
        compiler_params=pltpu.CompilerParams(use_tc_tiling_on_sc=False),
        out_type=jax.ShapeDtypeStruct((NW, n_groups, GROUP, embed), jnp.float32),
        scratch_types=[
            pltpu.VMEM((n_groups, GROUP), jnp.int32),
            pltpu.VMEM((2, K, GROUP, embed), jnp.float32),
            pltpu.SemaphoreType.DMA,
            pltpu.SemaphoreType.DMA,
        ],
    )
    def emb_kernel(idx_hbm, table_hbm, out_hbm, idx_v, rows_v, sem_g, sem_o):
        wid = lax.axis_index("s") * NC + lax.axis_index("c")
        pltpu.sync_copy(idx_hbm.at[wid], idx_v)

        def fire_gathers(h):
            s = lax.rem(h, 2)
            for j in range(K):
                pltpu.async_copy(
                    table_hbm.at[idx_v.at[h * K + j]], rows_v.at[s, j], sem_g
                )

        def fire_writes(h):
            s = lax.rem(h, 2)
            for j in range(K):
                pltpu.async_copy(
                    rows_v.at[s, j], out_hbm.at[wid, h * K + j], sem_o
                )

        def drain(sem, n):
            for _ in range(n):
                pltpu.make_async_copy(rows_v.at[0, 0], out_hbm.at[wid, 0], sem).wait()

        fire_gathers(0)
        drain(sem_g, K)
        fire_writes(0)
        if n_halves > 2:
            fire_gathers(1)

            @pl.loop(1, n_halves - 1)
            def _(h):
                drain(sem_g, K)
                drain(sem_o, K)
                fire_writes(h)
                fire_gathers(h + 1)

        else:
            fire_gathers(1)

        drain(sem_g, K)
        drain(sem_o, K)
        fire_writes(n_halves - 1)
        drain(sem_o, K)

    return emb_kernel


def kernel(x, vectors):
    batch, hist = x.shape
    vocab, embed = vectors.shape
    total = batch * hist
    idx = x.astype(jnp.int32).reshape(NW, total // (NW * GROUP), GROUP)
    out = _build(total, vocab, embed)(idx, vectors)
    return out.reshape(batch, hist, embed)


# speedup vs baseline: 1.4809x; 1.4809x over previous
import functools

import jax
import jax.numpy as jnp
from jax import lax
from jax.experimental import pallas as pl
from jax.experimental.pallas import tpu as pltpu
from jax.experimental.pallas import tpu_sc as plsc

NC = 2
NS = 16
NW = NC * NS
CB = 8
HP = 56
EP = 128


@functools.lru_cache(maxsize=None)
def _build(batch: int, hist: int, vocab: int, embed: int):
    assert batch % (NW * CB) == 0
    rows_per_w = batch // NW
    n_chunks = rows_per_w // CB
    mesh = plsc.VectorSubcoreMesh(core_axis_name="c", subcore_axis_name="s")

    @functools.partial(
        pl.kernel,
        mesh=mesh,
        compiler_params=pltpu.CompilerParams(use_tc_tiling_on_sc=False),
        out_type=jax.ShapeDtypeStruct((NW, n_chunks, CB, HP, EP), jnp.float32),
        scratch_types=[
            pltpu.VMEM((rows_per_w, hist), jnp.int32),
            pltpu.VMEM((2, CB, hist, embed), jnp.float32),
            pltpu.SemaphoreType.DMA,
            pltpu.SemaphoreType.DMA,
        ],
    )
    def emb_kernel(idx_hbm, table_hbm, out_hbm, idx_v, stage_v, sem_g, sem_o):
        wid = lax.axis_index("s") * NC + lax.axis_index("c")
        pltpu.sync_copy(idx_hbm.at[wid], idx_v)

        def fire_gathers(c):
            s = lax.rem(c, 2)
            for bb in range(CB):
                pltpu.async_copy(
                    table_hbm.at[idx_v.at[c * CB + bb]],
                    stage_v.at[s, bb],
                    sem_g,
                )

        def fire_write(c):
            s = lax.rem(c, 2)
            pltpu.async_copy(
                stage_v.at[s],
                out_hbm.at[wid, c, slice(None), pl.ds(0, hist), pl.ds(0, embed)],
                sem_o,
            )

        def drain_g(n):
            for _ in range(n):
                pltpu.make_async_copy(
                    out_hbm.at[wid, 0, 0, pl.ds(0, hist), pl.ds(0, embed)],
                    stage_v.at[0, 0],
                    sem_g,
                ).wait()

        def drain_o(n):
            for _ in range(n):
                pltpu.make_async_copy(
                    stage_v.at[0],
                    out_hbm.at[wid, 0, slice(None), pl.ds(0, hist), pl.ds(0, embed)],
                    sem_o,
                ).wait()

        fire_gathers(0)
        drain_g(CB)
        fire_write(0)
        fire_gathers(1)

        @pl.loop(1, n_chunks - 1)
        def _(c):
            drain_g(CB)
            drain_o(1)
            fire_write(c)
            fire_gathers(c + 1)

        drain_g(CB)
        drain_o(1)
        fire_write(n_chunks - 1)
        drain_o(1)

    return emb_kernel


def kernel(x, vectors):
    batch, hist = x.shape
    vocab, embed = vectors.shape
    idx = x.astype(jnp.int32).reshape(NW, batch // NW, hist)
    out = _build(batch, hist, vocab, embed)(idx, vectors)
    return out.reshape(batch, HP, EP)[:, :hist, :embed]
